# baseline (speedup 1.0000x reference)
import jax
import jax.numpy as jnp
from jax import lax
from jax.experimental import pallas as pl
from jax.experimental.pallas import tpu as pltpu
from jax.experimental.pallas import tpu_sc as plsc

NSTEP = 20
NFIELD = 26
NEMB = 64
P = NSTEP * NFIELD
BSZ = 1024
LANES = 16
CHUNK = 104
NCHUNK = P // CHUNK

NC = 2
NS = 16
NW = NC * NS
B_PER_W = BSZ // NW


def _body(x_hbm, feat_hbm, glob_hbm, out_hbm, glob_v, rows_v, idx_v, sem):
    wid = lax.axis_index("s") * NC + lax.axis_index("c")
    b0 = wid * B_PER_W

    pltpu.sync_copy(glob_hbm, glob_v)

    def per_batch(i, carry):
        b = b0 + i
        pltpu.sync_copy(x_hbm.at[b], idx_v)
        cps = [
            pltpu.async_copy(
                feat_hbm.at[idx_v.at[c]],
                rows_v.at[pl.ds(c * CHUNK, CHUNK)],
                sem,
            )
            for c in range(NCHUNK)
        ]
        for cp in cps:
            cp.wait()

        def add_row(p, c2):
            for s in range(NEMB // LANES):
                sl = pl.ds(s * LANES, LANES)
                rows_v[p, sl] = rows_v[p, sl] + glob_v[p, sl]
            return c2

        lax.fori_loop(0, P, add_row, 0)
        pltpu.sync_copy(rows_v, out_hbm.at[b])
        return carry

    lax.fori_loop(0, B_PER_W, per_batch, 0)


@jax.jit
def kernel(x, feat_table, global_table):
    x3 = x.reshape(BSZ, NCHUNK, CHUNK)
    mesh = plsc.VectorSubcoreMesh(core_axis_name="c", subcore_axis_name="s")
    run = pl.kernel(
        _body,
        out_type=jax.ShapeDtypeStruct((BSZ, P, NEMB), jnp.float32),
        mesh=mesh,
        compiler_params=pltpu.CompilerParams(use_tc_tiling_on_sc=False),
        scratch_types=[
            pltpu.VMEM((P, NEMB), jnp.float32),
            pltpu.VMEM((P, NEMB), jnp.float32),
            pltpu.VMEM((NCHUNK, CHUNK), jnp.int32),
            pltpu.SemaphoreType.DMA,
        ],
    )
    return run(x3, feat_table, global_table)


# speedup vs baseline: 1.1680x; 1.1680x over previous
import jax
import jax.numpy as jnp
from jax import lax
from jax.experimental import pallas as pl
from jax.experimental.pallas import tpu as pltpu
from jax.experimental.pallas import tpu_sc as plsc

NSTEP = 20
NFIELD = 26
NEMB = 64
P = NSTEP * NFIELD
BSZ = 1024
LANES = 16
CHUNK = 104
NCHUNK = P // CHUNK

NC = 2
NS = 16
NW = NC * NS
B_PER_W = BSZ // NW


def _body(x_hbm, feat_hbm, glob_hbm, out_hbm, glob_v, rows_v, idx_v, gsem, wsem):
    wid = lax.axis_index("s") * NC + lax.axis_index("c")
    b0 = wid * B_PER_W

    pltpu.sync_copy(glob_hbm, glob_v)

    def fire_gathers(slot, b):
        pltpu.sync_copy(x_hbm.at[b], idx_v.at[slot])
        for c in range(NCHUNK):
            pltpu.async_copy(
                feat_hbm.at[idx_v.at[slot, c]],
                rows_v.at[slot, pl.ds(c * CHUNK, CHUNK)],
                gsem.at[slot],
            )

    def wait_gathers(slot):
        for c in range(NCHUNK):
            pltpu.make_async_copy(
                feat_hbm.at[idx_v.at[slot, c]],
                rows_v.at[slot, pl.ds(c * CHUNK, CHUNK)],
                gsem.at[slot],
            ).wait()

    fire_gathers(0, b0)

    def per_batch(i, carry):
        slot = lax.rem(i, 2)
        nxt = 1 - slot

        @pl.when(i >= 1)
        def _():
            pltpu.make_async_copy(
                rows_v.at[nxt], out_hbm.at[b0 + i - 1], wsem.at[nxt]
            ).wait()

        @pl.when(i + 1 < B_PER_W)
        def _():
            fire_gathers(nxt, b0 + i + 1)

        wait_gathers(slot)

        def add_row(p, c2):
            for s in range(NEMB // LANES):
                sl = pl.ds(s * LANES, LANES)
                rows_v[slot, p, sl] = rows_v[slot, p, sl] + glob_v[p, sl]
            return c2

        lax.fori_loop(0, P, add_row, 0)
        pltpu.async_copy(rows_v.at[slot], out_hbm.at[b0 + i], wsem.at[slot])
        return carry

    lax.fori_loop(0, B_PER_W, per_batch, 0)

    last = B_PER_W - 1
    pltpu.make_async_copy(
        rows_v.at[last % 2], out_hbm.at[b0 + last], wsem.at[last % 2]
    ).wait()


@jax.jit
def kernel(x, feat_table, global_table):
    x3 = x.reshape(BSZ, NCHUNK, CHUNK)
    mesh = plsc.VectorSubcoreMesh(core_axis_name="c", subcore_axis_name="s")
    run = pl.kernel(
        _body,
        out_type=jax.ShapeDtypeStruct((BSZ, P, NEMB), jnp.float32),
        mesh=mesh,
        compiler_params=pltpu.CompilerParams(use_tc_tiling_on_sc=False),
        scratch_types=[
            pltpu.VMEM((P, NEMB), jnp.float32),
            pltpu.VMEM((2, P, NEMB), jnp.float32),
            pltpu.VMEM((2, NCHUNK, CHUNK), jnp.int32),
            pltpu.SemaphoreType.DMA((2,)),
            pltpu.SemaphoreType.DMA((2,)),
        ],
    )
    return run(x3, feat_table, global_table)
